# baseline (speedup 1.0000x reference)
import functools

import jax
import jax.numpy as jnp
import numpy as np
from jax.experimental import pallas as pl

B = 10000
S = 8
N_IN = 256
N_H = 64

B_BLK = 400
SUB = 16
N_SUB = B_BLK // SUB
GRID = B // B_BLK

_blk = np.arange(128) // 8
_MASK = (_blk[:, None] == _blk[None, :]).astype(np.float32)

_SEL = np.zeros((48, 128), dtype=np.float32)
for _i in range(16):
    _SEL[_i, _i * 8:_i * 8 + 7] = 1.0 / 7.0
    _SEL[16 + _i, _i * 8 + 7] = 1.0
    _SEL[32 + _i, _i * 8 + 6] = 1.0
_SEL = jnp.asarray(_SEL)
_MASK = jnp.asarray(_MASK)


def _stage1_body(seq_ref, adj_ref, wcp_ref, mask_ref, sel_ref, wkbd_ref,
                 bias_ref, slope_ref, out_ref):
    x = seq_ref[...].reshape(B_BLK * S, N_IN)
    fts = jnp.dot(x, wcp_ref[...], preferred_element_type=jnp.float32)
    adjf = adj_ref[...].reshape(B_BLK * S, S)
    mask = mask_ref[...]
    sel = sel_ref[...]
    wkbd = wkbd_ref[...]
    bias = bias_ref[...]
    slope = slope_ref[...]
    for j in range(N_SUB):
        rows = slice(j * 128, (j + 1) * 128)
        sub = adjf[rows, :]
        bd = jnp.concatenate([sub] * 16, axis=1) * mask
        h = jnp.dot(bd, fts[rows, :], preferred_element_type=jnp.float32)
        y = h + bias
        hact = jnp.where(y >= 0, y, slope * y)
        r = jnp.dot(sel, hact, preferred_element_type=jnp.float32)
        z12 = jnp.dot(r[16:32, :], wkbd, preferred_element_type=jnp.float32)
        o = slice(j * SUB, (j + 1) * SUB)
        out_ref[o, 0:64] = r[0:16, 0:64]
        out_ref[o, 64:192] = z12
        out_ref[o, 192:256] = r[32:48, 64:128]


def _stage2_body(s_ref, bk_ref, r1_ref, r2_ref):
    x = s_ref[...]
    c = x[:, 0:64]
    z1 = x[:, 64:128]
    z2 = x[:, 128:192]
    han = x[:, 192:256]
    bkc = bk_ref[0, 0]
    bkp = bk_ref[0, 1]
    s0 = jnp.sum(z1 * c, axis=1, keepdims=True) + bkc
    cr = jnp.concatenate([c[B - 2:B - 1, :], c[:B - 1, :]], axis=0)
    s1 = jnp.sum(z1 * cr, axis=1, keepdims=True) + bkc
    r1_ref[...] = jnp.concatenate([s0, s1], axis=0)
    t0 = jnp.sum(z2 * han, axis=1, keepdims=True) + bkp
    hr = jnp.concatenate([han[B - 2:B - 1, :], han[:B - 1, :]], axis=0)
    t1 = jnp.sum(z2 * hr, axis=1, keepdims=True) + bkp
    r2_ref[...] = jnp.concatenate([t0, t1], axis=0)


@functools.partial(jax.jit, static_argnames=("interpret",))
def _run(seq1, adj, Wc, bc, a_c, Wp, bp, a_p, Wk_c, bk_c, Wk_p, bk_p,
         interpret=False):
    wcp = jnp.concatenate([Wc.T, Wp.T], axis=1)
    wkbd = jnp.zeros((128, 128), jnp.float32)
    wkbd = wkbd.at[0:64, 0:64].set(Wk_c).at[64:128, 64:128].set(Wk_p)
    bias = jnp.concatenate([bc, bp])[None, :]
    slope = jnp.concatenate([jnp.broadcast_to(a_c, (64,)),
                             jnp.broadcast_to(a_p, (64,))])[None, :]
    bk = jnp.stack([bk_c[0], bk_p[0]])[None, :]

    stage1 = pl.pallas_call(
        _stage1_body,
        grid=(GRID,),
        in_specs=[
            pl.BlockSpec((B_BLK, S, N_IN), lambda i: (i, 0, 0)),
            pl.BlockSpec((B_BLK, S, S), lambda i: (i, 0, 0)),
            pl.BlockSpec((N_IN, 128), lambda i: (0, 0)),
            pl.BlockSpec((128, 128), lambda i: (0, 0)),
            pl.BlockSpec((48, 128), lambda i: (0, 0)),
            pl.BlockSpec((128, 128), lambda i: (0, 0)),
            pl.BlockSpec((1, 128), lambda i: (0, 0)),
            pl.BlockSpec((1, 128), lambda i: (0, 0)),
        ],
        out_specs=pl.BlockSpec((B_BLK, 256), lambda i: (i, 0)),
        out_shape=jax.ShapeDtypeStruct((B, 256), jnp.float32),
        interpret=interpret,
    )(seq1, adj, wcp, _MASK, _SEL, wkbd, bias, slope)

    ret1, ret2 = pl.pallas_call(
        _stage2_body,
        out_shape=(jax.ShapeDtypeStruct((2 * B, 1), jnp.float32),
                   jax.ShapeDtypeStruct((2 * B, 1), jnp.float32)),
        interpret=interpret,
    )(stage1, bk)
    return ret1, ret2


def kernel(seq1, adj, Wc, bc, a_c, Wp, bp, a_p, Wk_c, bk_c, Wk_p, bk_p):
    return _run(seq1, adj, Wc, bc, a_c, Wp, bp, a_p,
                Wk_c, bk_c, Wk_p, bk_p)
